# Initial kernel scaffold; baseline (speedup 1.0000x reference)
import jax
import jax.numpy as jnp
from jax.experimental import pallas as pl


def kernel(x, edge_index, W1, b1, gamma, beta, W2, b2):
    raise NotImplementedError("write your pallas kernel here")


# speedup vs baseline: 30.0439x; 30.0439x over previous
import functools

import jax
import jax.numpy as jnp
from jax import lax
from jax.experimental import pallas as pl
from jax.experimental.pallas import tpu as pltpu
from jax.experimental.pallas import tpu_sc as plsc

_N = 10000
_D = 128
_H = 16
_C = 40

_NP = 10112
_NSUB = 16
_NCORE = 2
_NW = _NCORE * _NSUB
_RPT = _NP // _NSUB
_CHUNK = 128

_SELU_ALPHA = 1.6732632423543772
_SELU_SCALE = 1.0507009873554805


def _sc_degree(cpt):
    mesh = plsc.VectorSubcoreMesh(core_axis_name="c", subcore_axis_name="s")

    @functools.partial(
        pl.kernel,
        out_type=jax.ShapeDtypeStruct((_NCORE, _NP, _H), jnp.float32),
        mesh=mesh,
        scratch_types=[
            pltpu.VMEM((cpt, _CHUNK), jnp.int32),
            pltpu.VMEM((_CHUNK, _H), jnp.float32),
            pltpu.VMEM_SHARED((_NP, _H), jnp.float32),
        ],
        compiler_params=pltpu.CompilerParams(use_tc_tiling_on_sc=False),
    )
    def deg_kernel(dst_hbm, ones_hbm, zeros_hbm, out_hbm, dst_v, ones_v, acc):
        c = lax.axis_index("c")
        s = lax.axis_index("s")
        wid = s * _NCORE + c
        pltpu.sync_copy(zeros_hbm.at[pl.ds(s * _RPT, _RPT)],
                        acc.at[pl.ds(s * _RPT, _RPT)])
        pltpu.sync_copy(ones_hbm, ones_v)
        pltpu.sync_copy(dst_hbm.at[wid], dst_v)
        plsc.subcore_barrier()

        @pl.loop(0, cpt)
        def _(j):
            pltpu.sync_copy(ones_v, acc.at[dst_v.at[j]], add=True)

        plsc.subcore_barrier()
        pltpu.sync_copy(acc.at[pl.ds(s * _RPT, _RPT)],
                        out_hbm.at[c, pl.ds(s * _RPT, _RPT)])

    return deg_kernel


def _sc_prop(cpt):
    mesh = plsc.VectorSubcoreMesh(core_axis_name="c", subcore_axis_name="s")

    @functools.partial(
        pl.kernel,
        out_type=jax.ShapeDtypeStruct((_NCORE, _NP, _H), jnp.float32),
        mesh=mesh,
        scratch_types=[
            pltpu.VMEM((cpt, _CHUNK), jnp.int32),
            pltpu.VMEM((cpt, _CHUNK), jnp.int32),
            pltpu.VMEM((_CHUNK, _H), jnp.float32),
            pltpu.VMEM((_CHUNK, _H), jnp.float32),
            pltpu.VMEM_SHARED((_NP, _H), jnp.float32),
            pltpu.SemaphoreType.DMA,
            pltpu.SemaphoreType.DMA,
        ],
        compiler_params=pltpu.CompilerParams(use_tc_tiling_on_sc=False),
    )
    def prop_kernel(u_hbm, src_hbm, dst_hbm, zeros_hbm, out_hbm,
                    src_v, dst_v, buf_a, buf_b, acc, sem_a, sem_b):
        c = lax.axis_index("c")
        s = lax.axis_index("s")
        wid = s * _NCORE + c
        pltpu.sync_copy(zeros_hbm.at[pl.ds(s * _RPT, _RPT)],
                        acc.at[pl.ds(s * _RPT, _RPT)])
        pltpu.sync_copy(src_hbm.at[wid], src_v)
        pltpu.sync_copy(dst_hbm.at[wid], dst_v)
        plsc.subcore_barrier()

        @pl.loop(0, cpt // 2)
        def _(j):
            j0 = j * 2
            j1 = j0 + 1
            ga = pltpu.async_copy(u_hbm.at[src_v.at[j0]], buf_a, sem_a)
            gb = pltpu.async_copy(u_hbm.at[src_v.at[j1]], buf_b, sem_b)
            ga.wait()
            pltpu.sync_copy(buf_a, acc.at[dst_v.at[j0]], add=True)
            gb.wait()
            pltpu.sync_copy(buf_b, acc.at[dst_v.at[j1]], add=True)

        plsc.subcore_barrier()
        pltpu.sync_copy(acc.at[pl.ds(s * _RPT, _RPT)],
                        out_hbm.at[c, pl.ds(s * _RPT, _RPT)])

    return prop_kernel


def _row_mask(val, other=0.0):
    rows = lax.broadcasted_iota(jnp.int32, (_NP, _H), 0)
    return jnp.where(rows < _N, val, other)


def _prep_body(xp_ref, w1_ref, degp_ref, dinv_ref, u0_ref):
    h0 = jnp.dot(xp_ref[...], w1_ref[...], preferred_element_type=jnp.float32)
    deg = degp_ref[0] + degp_ref[1] + 1.0
    dinv = lax.rsqrt(deg)
    dinv_ref[...] = dinv
    u0_ref[...] = _row_mask(dinv * h0)


def _mid_body(p_ref, u_ref, dinv_ref, o_ref):
    t = p_ref[0] + p_ref[1] + u_ref[...]
    d = dinv_ref[...]
    o_ref[...] = _row_mask(d * d * t)


def _bn_body(p_ref, u_ref, dinv_ref, b1_ref, gamma_ref, beta_ref, o_ref):
    t = p_ref[0] + p_ref[1] + u_ref[...]
    d = dinv_ref[...]
    h = d * t + b1_ref[...]
    hm = _row_mask(h)
    mean = jnp.sum(hm, axis=0, keepdims=True) * (1.0 / _N)
    dev = _row_mask(h - mean)
    var = jnp.sum(dev * dev, axis=0, keepdims=True) * (1.0 / _N)
    hn = (h - mean) * lax.rsqrt(var + 1e-5) * gamma_ref[...] + beta_ref[...]
    sel = _SELU_SCALE * jnp.where(hn > 0, hn, _SELU_ALPHA * (jnp.exp(hn) - 1.0))
    o_ref[...] = _row_mask(d * sel)


def _fin_body(p_ref, u_ref, dinv_ref, w2_ref, b2_ref, o_ref):
    t = p_ref[0] + p_ref[1] + u_ref[...]
    g = (dinv_ref[...] * t)[:_N]
    z = jnp.dot(g, w2_ref[...], preferred_element_type=jnp.float32) + b2_ref[...]
    m = jnp.max(z, axis=1, keepdims=True)
    e = jnp.exp(z - m)
    o_ref[...] = (z - m) - jnp.log(jnp.sum(e, axis=1, keepdims=True))


def _sds(shape):
    return jax.ShapeDtypeStruct(shape, jnp.float32)


def kernel(x, edge_index, W1, b1, gamma, beta, W2, b2):
    e = edge_index.shape[1]
    unit = _NW * _CHUNK * 2
    ep = ((e + unit - 1) // unit) * unit
    cpt = ep // (_NW * _CHUNK)
    pad = ep - e

    fill = jnp.full((pad,), _N, jnp.int32)
    src3 = jnp.concatenate([edge_index[0], fill]).reshape(_NW, cpt, _CHUNK)
    dst3 = jnp.concatenate([edge_index[1], fill]).reshape(_NW, cpt, _CHUNK)
    xp = jnp.pad(x, ((0, _NP - _N), (0, 0)))
    zeros = jnp.zeros((_NP, _H), jnp.float32)
    ones = jnp.ones((_CHUNK, _H), jnp.float32)
    b1r = b1.reshape(1, _H)
    gammar = gamma.reshape(1, _H)
    betar = beta.reshape(1, _H)
    b2r = b2.reshape(1, _C)

    degp = _sc_degree(cpt)(dst3, ones, zeros)
    dinv, u0 = pl.pallas_call(
        _prep_body, out_shape=(_sds((_NP, _H)), _sds((_NP, _H))))(xp, W1, degp)

    prop = _sc_prop(cpt)
    p1 = prop(u0, src3, dst3, zeros)
    u1 = pl.pallas_call(_mid_body, out_shape=_sds((_NP, _H)))(p1, u0, dinv)
    p2 = prop(u1, src3, dst3, zeros)
    u2 = pl.pallas_call(_bn_body, out_shape=_sds((_NP, _H)))(
        p2, u1, dinv, b1r, gammar, betar)
    p3 = prop(u2, src3, dst3, zeros)
    out = pl.pallas_call(_fin_body, out_shape=_sds((_N, _C)))(
        p3, u2, dinv, W2, b2r)
    return out


# speedup vs baseline: 34.2916x; 1.1414x over previous
import functools

import jax
import jax.numpy as jnp
from jax import lax
from jax.experimental import pallas as pl
from jax.experimental.pallas import tpu as pltpu
from jax.experimental.pallas import tpu_sc as plsc

_N = 10000
_D = 128
_H = 16
_C = 40

_NP = 10112
_NSUB = 16
_NCORE = 2
_NW = _NCORE * _NSUB
_RPT = _NP // _NSUB
_CHUNK = 128
_NB = 4

_SELU_ALPHA = 1.6732632423543772
_SELU_SCALE = 1.0507009873554805


def _sc_degree(cpt):
    mesh = plsc.VectorSubcoreMesh(core_axis_name="c", subcore_axis_name="s")

    @functools.partial(
        pl.kernel,
        out_type=jax.ShapeDtypeStruct((_NCORE, _NP, _H), jnp.float32),
        mesh=mesh,
        scratch_types=[
            pltpu.VMEM((cpt, _CHUNK), jnp.int32),
            pltpu.VMEM((_CHUNK, _H), jnp.float32),
            pltpu.VMEM_SHARED((_NP, _H), jnp.float32),
            [pltpu.SemaphoreType.DMA] * _NB,
        ],
        compiler_params=pltpu.CompilerParams(use_tc_tiling_on_sc=False),
    )
    def deg_kernel(dst_hbm, ones_hbm, zeros_hbm, out_hbm, dst_v, ones_v, acc,
                   sems):
        c = lax.axis_index("c")
        s = lax.axis_index("s")
        wid = s * _NCORE + c
        pltpu.sync_copy(zeros_hbm.at[pl.ds(s * _RPT, _RPT)],
                        acc.at[pl.ds(s * _RPT, _RPT)])
        pltpu.sync_copy(ones_hbm, ones_v)
        pltpu.sync_copy(dst_hbm.at[wid], dst_v)
        plsc.subcore_barrier()

        for b in range(_NB):
            pltpu.async_copy(ones_v, acc.at[dst_v.at[b]], sems[b], add=True)

        @pl.loop(1, cpt // _NB)
        def _(k):
            for b in range(_NB):
                j = k * _NB + b
                pltpu.make_async_copy(ones_v, acc.at[dst_v.at[j]],
                                      sems[b]).wait()
                pltpu.async_copy(ones_v, acc.at[dst_v.at[j]], sems[b],
                                 add=True)

        for b in range(_NB):
            pltpu.make_async_copy(ones_v, acc.at[dst_v.at[b]], sems[b]).wait()

        plsc.subcore_barrier()
        pltpu.sync_copy(acc.at[pl.ds(s * _RPT, _RPT)],
                        out_hbm.at[c, pl.ds(s * _RPT, _RPT)])

    return deg_kernel


def _sc_prop(cpt):
    mesh = plsc.VectorSubcoreMesh(core_axis_name="c", subcore_axis_name="s")

    @functools.partial(
        pl.kernel,
        out_type=jax.ShapeDtypeStruct((_NCORE, _NP, _H), jnp.float32),
        mesh=mesh,
        scratch_types=[
            pltpu.VMEM((cpt, _CHUNK), jnp.int32),
            pltpu.VMEM((cpt, _CHUNK), jnp.int32),
            [pltpu.VMEM((_CHUNK, _H), jnp.float32)] * _NB,
            [pltpu.SemaphoreType.DMA] * _NB,
            [pltpu.SemaphoreType.DMA] * _NB,
            pltpu.VMEM_SHARED((_NP, _H), jnp.float32),
        ],
        compiler_params=pltpu.CompilerParams(use_tc_tiling_on_sc=False),
    )
    def prop_kernel(u_hbm, src_hbm, dst_hbm, zeros_hbm, out_hbm,
                    src_v, dst_v, bufs, gsems, ssems, acc):
        c = lax.axis_index("c")
        s = lax.axis_index("s")
        wid = s * _NCORE + c
        pltpu.sync_copy(zeros_hbm.at[pl.ds(s * _RPT, _RPT)],
                        acc.at[pl.ds(s * _RPT, _RPT)])
        pltpu.sync_copy(src_hbm.at[wid], src_v)
        pltpu.sync_copy(dst_hbm.at[wid], dst_v)
        plsc.subcore_barrier()

        for b in range(_NB):
            pltpu.async_copy(u_hbm.at[src_v.at[b]], bufs[b], gsems[b])

        @pl.loop(0, cpt // _NB - 1)
        def _(k):
            for b in range(_NB):
                j = k * _NB + b
                pltpu.make_async_copy(u_hbm.at[src_v.at[j]], bufs[b],
                                      gsems[b]).wait()
                pltpu.async_copy(bufs[b], acc.at[dst_v.at[j]], ssems[b],
                                 add=True)
            for b in range(_NB):
                j = k * _NB + b
                pltpu.make_async_copy(bufs[b], acc.at[dst_v.at[j]],
                                      ssems[b]).wait()
                pltpu.async_copy(u_hbm.at[src_v.at[j + _NB]], bufs[b],
                                 gsems[b])

        for b in range(_NB):
            j = cpt - _NB + b
            pltpu.make_async_copy(u_hbm.at[src_v.at[j]], bufs[b],
                                  gsems[b]).wait()
            pltpu.async_copy(bufs[b], acc.at[dst_v.at[j]], ssems[b], add=True)
        for b in range(_NB):
            j = cpt - _NB + b
            pltpu.make_async_copy(bufs[b], acc.at[dst_v.at[j]],
                                  ssems[b]).wait()

        plsc.subcore_barrier()
        pltpu.sync_copy(acc.at[pl.ds(s * _RPT, _RPT)],
                        out_hbm.at[c, pl.ds(s * _RPT, _RPT)])

    return prop_kernel


def _row_mask(val, other=0.0):
    rows = lax.broadcasted_iota(jnp.int32, (_NP, _H), 0)
    return jnp.where(rows < _N, val, other)


def _prep_body(xp_ref, w1_ref, degp_ref, dinv_ref, u0_ref):
    h0 = jnp.dot(xp_ref[...], w1_ref[...], preferred_element_type=jnp.float32)
    deg = degp_ref[0] + degp_ref[1] + 1.0
    dinv = lax.rsqrt(deg)
    dinv_ref[...] = dinv
    u0_ref[...] = _row_mask(dinv * h0)


def _mid_body(p_ref, u_ref, dinv_ref, o_ref):
    t = p_ref[0] + p_ref[1] + u_ref[...]
    d = dinv_ref[...]
    o_ref[...] = _row_mask(d * d * t)


def _bn_body(p_ref, u_ref, dinv_ref, b1_ref, gamma_ref, beta_ref, o_ref):
    t = p_ref[0] + p_ref[1] + u_ref[...]
    d = dinv_ref[...]
    h = d * t + b1_ref[...]
    hm = _row_mask(h)
    mean = jnp.sum(hm, axis=0, keepdims=True) * (1.0 / _N)
    dev = _row_mask(h - mean)
    var = jnp.sum(dev * dev, axis=0, keepdims=True) * (1.0 / _N)
    hn = (h - mean) * lax.rsqrt(var + 1e-5) * gamma_ref[...] + beta_ref[...]
    sel = _SELU_SCALE * jnp.where(hn > 0, hn, _SELU_ALPHA * (jnp.exp(hn) - 1.0))
    o_ref[...] = _row_mask(d * sel)


def _fin_body(p_ref, u_ref, dinv_ref, w2_ref, b2_ref, o_ref):
    t = p_ref[0] + p_ref[1] + u_ref[...]
    g = (dinv_ref[...] * t)[:_N]
    z = jnp.dot(g, w2_ref[...], preferred_element_type=jnp.float32) + b2_ref[...]
    m = jnp.max(z, axis=1, keepdims=True)
    e = jnp.exp(z - m)
    o_ref[...] = (z - m) - jnp.log(jnp.sum(e, axis=1, keepdims=True))


def _sds(shape):
    return jax.ShapeDtypeStruct(shape, jnp.float32)


def kernel(x, edge_index, W1, b1, gamma, beta, W2, b2):
    e = edge_index.shape[1]
    unit = _NW * _CHUNK * 2
    ep = ((e + unit - 1) // unit) * unit
    cpt = ep // (_NW * _CHUNK)
    pad = ep - e

    fill = jnp.full((pad,), _N, jnp.int32)
    src3 = jnp.concatenate([edge_index[0], fill]).reshape(_NW, cpt, _CHUNK)
    dst3 = jnp.concatenate([edge_index[1], fill]).reshape(_NW, cpt, _CHUNK)
    xp = jnp.pad(x, ((0, _NP - _N), (0, 0)))
    zeros = jnp.zeros((_NP, _H), jnp.float32)
    ones = jnp.ones((_CHUNK, _H), jnp.float32)
    b1r = b1.reshape(1, _H)
    gammar = gamma.reshape(1, _H)
    betar = beta.reshape(1, _H)
    b2r = b2.reshape(1, _C)

    degp = _sc_degree(cpt)(dst3, ones, zeros)
    dinv, u0 = pl.pallas_call(
        _prep_body, out_shape=(_sds((_NP, _H)), _sds((_NP, _H))))(xp, W1, degp)

    prop = _sc_prop(cpt)
    p1 = prop(u0, src3, dst3, zeros)
    u1 = pl.pallas_call(_mid_body, out_shape=_sds((_NP, _H)))(p1, u0, dinv)
    p2 = prop(u1, src3, dst3, zeros)
    u2 = pl.pallas_call(_bn_body, out_shape=_sds((_NP, _H)))(
        p2, u1, dinv, b1r, gammar, betar)
    p3 = prop(u2, src3, dst3, zeros)
    out = pl.pallas_call(_fin_body, out_shape=_sds((_N, _C)))(
        p3, u2, dinv, W2, b2r)
    return out
